# baseline (speedup 1.0000x reference)
import jax
import jax.numpy as jnp
from jax.experimental import pallas as pl

NUM_HEADS = 16
D_HEAD = 64
D_MODEL = NUM_HEADS * D_HEAD
D_LATENT = 128
TOP_K = 512
SEQ = 2048
BM = 256
NEG = -1e9


def _dot(a, b, trans_b=False):
    dn = (((1,), (1 if trans_b else 0,)), ((), ()))
    return jax.lax.dot_general(a, b, dn, preferred_element_type=jnp.float32)


def _proj_kernel(x_ref, wq_ref, bq_ref, wkv_ref, bkv_ref, wkup_ref, bkup_ref,
                 wvup_ref, bvup_ref, wqidx_ref, bqidx_ref,
                 q_ref, qidx_ref, kdown_ref, kup_ref, vup_ref):
    x = x_ref[...]
    xb = x.astype(jnp.bfloat16)
    scale = 1.0 / (D_HEAD ** 0.5)
    q_ref[...] = ((_dot(xb, wq_ref[...]) + bq_ref[...]) * scale
                  ).astype(jnp.bfloat16)
    qidx_ref[...] = _dot(x, wqidx_ref[...]) + bqidx_ref[...]
    kv = _dot(x, wkv_ref[...]) + bkv_ref[...]
    kd = kv[:, :D_LATENT]
    vd = kv[:, D_LATENT:]
    kdown_ref[...] = kd
    kup_ref[...] = (_dot(kd.astype(jnp.bfloat16), wkup_ref[...])
                    + bkup_ref[...]).astype(jnp.bfloat16)
    vup_ref[...] = (_dot(vd.astype(jnp.bfloat16), wvup_ref[...])
                    + bvup_ref[...]).astype(jnp.bfloat16)


def _make_attn_kernel(s_kv, row_base, bm):
    def attn_kernel(qidx_ref, kdown_ref, q_ref, kup_ref, vup_ref, wout_ref,
                    bout_ref, out_ref):
        qb = pl.program_id(0)
        rows = (row_base + qb * bm
                + jax.lax.broadcasted_iota(jnp.int32, (bm, s_kv), 0))
        cols = jax.lax.broadcasted_iota(jnp.int32, (bm, s_kv), 1)
        causal = cols <= rows

        if s_kv <= TOP_K:
            bias = jnp.where(causal, 0.0, NEG)
        else:
            qi = qidx_ref[...]
            kd = kdown_ref[...]
            fuzzy = jnp.maximum(_dot(qi, kd, trans_b=True), 0.0)
            vi = jnp.where(causal,
                           jax.lax.bitcast_convert_type(fuzzy, jnp.int32), -1)

            lo = jnp.full((bm, 1), -2, jnp.int32)
            hi = jnp.full((bm, 1), 0x7F800000, jnp.int32)
            for _ in range(31):
                mid = lo + (hi - lo) // 2
                cnt = jnp.sum(jnp.where(vi > mid, 1.0, 0.0),
                              axis=1, keepdims=True)
                small = cnt < TOP_K
                hi = jnp.where(small, mid, hi)
                lo = jnp.where(small, lo, mid)
            cut = hi

            gt = vi > cut
            cnt_gt = jnp.sum(jnp.where(gt, 1.0, 0.0), axis=1, keepdims=True)
            rem = TOP_K - cnt_gt
            eqcol = jnp.where((vi == cut) & (vi >= 0), cols,
                              jnp.int32(1 << 30))

            lo2 = jnp.full((bm, 1), -1, jnp.int32)
            hi2 = jnp.full((bm, 1), s_kv - 1, jnp.int32)
            for _ in range(11):
                mid = lo2 + (hi2 - lo2) // 2
                cnt = jnp.sum(jnp.where(eqcol <= mid, 1.0, 0.0),
                              axis=1, keepdims=True)
                enough = cnt >= rem
                hi2 = jnp.where(enough, mid, hi2)
                lo2 = jnp.where(enough, lo2, mid)

            selected = gt | ((eqcol <= hi2) & (rem > 0))
            bias = jnp.where(selected, 0.0, NEG)

        q = q_ref[...]
        ku = kup_ref[...]
        vu = vup_ref[...]
        ctx = []
        for h in range(NUM_HEADS):
            sl = slice(h * D_HEAD, (h + 1) * D_HEAD)
            s = _dot(q[:, sl], ku[:, sl], trans_b=True) + bias
            e = jnp.exp(s)
            denom = jnp.sum(e, axis=1, keepdims=True)
            cu = _dot(e.astype(jnp.bfloat16), vu[:, sl])
            ctx.append(cu * (1.0 / denom))
        ctx = jnp.concatenate(ctx, axis=1)
        out_ref[...] = (_dot(ctx.astype(jnp.bfloat16), wout_ref[...])
                        + bout_ref[...])

    return attn_kernel


def kernel(x, Wq, bq, Wkv_down, bkv_down, Wk_up, bk_up, Wv_up, bv_up,
           Wq_idx, bq_idx, Wout, bout):
    b, s, dm = x.shape
    x2 = x.reshape(s, dm)
    Wq = Wq.astype(jnp.bfloat16)
    Wk_up = Wk_up.astype(jnp.bfloat16)
    Wv_up = Wv_up.astype(jnp.bfloat16)
    Wout = Wout.astype(jnp.bfloat16)
    row_blk = lambda i: (i, 0)
    whole = lambda i: (0, 0)

    def full_spec(arr):
        return pl.BlockSpec(arr.shape, whole)

    b2 = lambda v: v.reshape(1, -1)

    q, qidx, kdown, kup, vup = pl.pallas_call(
        _proj_kernel,
        grid=(s // BM,),
        in_specs=[
            pl.BlockSpec((BM, dm), row_blk),
            full_spec(Wq), pl.BlockSpec((1, dm), whole),
            full_spec(Wkv_down), pl.BlockSpec((1, 2 * D_LATENT), whole),
            full_spec(Wk_up), pl.BlockSpec((1, dm), whole),
            full_spec(Wv_up), pl.BlockSpec((1, dm), whole),
            full_spec(Wq_idx), pl.BlockSpec((1, D_LATENT), whole),
        ],
        out_specs=[
            pl.BlockSpec((BM, dm), row_blk),
            pl.BlockSpec((BM, D_LATENT), row_blk),
            pl.BlockSpec((BM, D_LATENT), row_blk),
            pl.BlockSpec((BM, dm), row_blk),
            pl.BlockSpec((BM, dm), row_blk),
        ],
        out_shape=[
            jax.ShapeDtypeStruct((s, dm), jnp.bfloat16),
            jax.ShapeDtypeStruct((s, D_LATENT), jnp.float32),
            jax.ShapeDtypeStruct((s, D_LATENT), jnp.float32),
            jax.ShapeDtypeStruct((s, dm), jnp.bfloat16),
            jax.ShapeDtypeStruct((s, dm), jnp.bfloat16),
        ],
    )(x2, Wq, b2(bq), Wkv_down, b2(bkv_down), Wk_up, b2(bk_up),
      Wv_up, b2(bv_up), Wq_idx, b2(bq_idx))

    outs = []
    bm2 = 256
    for g in range(8):
        row_base = g * bm2
        s_kv = (g + 1) * bm2
        grp_blk = lambda i, g=g: (i + g, 0)
        out_g = pl.pallas_call(
            _make_attn_kernel(s_kv, row_base, bm2),
            grid=(1,),
            in_specs=[
                pl.BlockSpec((bm2, D_LATENT), grp_blk),
                pl.BlockSpec((s_kv, D_LATENT), whole),
                pl.BlockSpec((bm2, dm), grp_blk),
                pl.BlockSpec((s_kv, dm), whole),
                pl.BlockSpec((s_kv, dm), whole),
                full_spec(Wout), pl.BlockSpec((1, dm), whole),
            ],
            out_specs=pl.BlockSpec((bm2, dm), row_blk),
            out_shape=jax.ShapeDtypeStruct((bm2, dm), jnp.float32),
        )(qidx, kdown, q, kup, vup, Wout, b2(bout))
        outs.append(out_g)

    return jnp.concatenate(outs, axis=0).reshape(b, s, dm)


# speedup vs baseline: 1.0860x; 1.0860x over previous
import jax
import jax.numpy as jnp
from jax.experimental import pallas as pl

NUM_HEADS = 16
D_HEAD = 64
D_MODEL = NUM_HEADS * D_HEAD
D_LATENT = 128
TOP_K = 512
SEQ = 2048
BM = 256
NEG = -1e9


def _dot(a, b, trans_b=False):
    dn = (((1,), (1 if trans_b else 0,)), ((), ()))
    return jax.lax.dot_general(a, b, dn, preferred_element_type=jnp.float32)


def _proj_kernel(x_ref, wq_ref, bq_ref, wkv_ref, bkv_ref, wkup_ref, bkup_ref,
                 wvup_ref, bvup_ref, wqidx_ref, bqidx_ref,
                 q_ref, qidx_ref, kdown_ref, kup_ref, vup_ref):
    x = x_ref[...]
    xb = x.astype(jnp.bfloat16)
    scale = 1.0 / (D_HEAD ** 0.5)
    q_ref[...] = ((_dot(xb, wq_ref[...]) + bq_ref[...]) * scale
                  ).astype(jnp.bfloat16)
    qidx_ref[...] = _dot(x, wqidx_ref[...]) + bqidx_ref[...]
    kv = _dot(x, wkv_ref[...]) + bkv_ref[...]
    kd = kv[:, :D_LATENT]
    vd = kv[:, D_LATENT:]
    kdown_ref[...] = kd
    kup_ref[...] = (_dot(kd.astype(jnp.bfloat16), wkup_ref[...])
                    + bkup_ref[...]).astype(jnp.bfloat16)
    vup_ref[...] = (_dot(vd.astype(jnp.bfloat16), wvup_ref[...])
                    + bvup_ref[...]).astype(jnp.bfloat16)


def _make_attn_kernel(s_kv, row_base, bm):
    def attn_kernel(qidx_ref, kdown_ref, q_ref, kup_ref, vup_ref, wout_ref,
                    bout_ref, out_ref):
        qb = pl.program_id(0)
        rows = (row_base + qb * bm
                + jax.lax.broadcasted_iota(jnp.int32, (bm, s_kv), 0))
        cols = jax.lax.broadcasted_iota(jnp.int32, (bm, s_kv), 1)
        causal = cols <= rows

        if s_kv <= TOP_K:
            bias = jnp.where(causal, 0.0, NEG)
        else:
            qi = qidx_ref[...]
            kd = kdown_ref[...]
            fuzzy = jnp.maximum(_dot(qi, kd, trans_b=True), 0.0)
            vi = jnp.where(causal,
                           jax.lax.bitcast_convert_type(fuzzy, jnp.int32), -1)

            lo = jnp.full((bm, 1), -2, jnp.int32)
            hi = jnp.full((bm, 1), 0x7F800000, jnp.int32)
            for _ in range(31):
                mid = lo + (hi - lo) // 2
                cnt = jnp.sum(jnp.where(vi > mid, 1.0, 0.0),
                              axis=1, keepdims=True)
                small = cnt < TOP_K
                hi = jnp.where(small, mid, hi)
                lo = jnp.where(small, lo, mid)
            cut = hi

            gt = vi > cut
            cnt_gt = jnp.sum(jnp.where(gt, 1.0, 0.0), axis=1, keepdims=True)
            rem = TOP_K - cnt_gt
            eqcol = jnp.where((vi == cut) & (vi >= 0), cols,
                              jnp.int32(1 << 30))

            lo2 = jnp.full((bm, 1), -1, jnp.int32)
            hi2 = jnp.full((bm, 1), s_kv - 1, jnp.int32)
            for _ in range(11):
                mid = lo2 + (hi2 - lo2) // 2
                cnt = jnp.sum(jnp.where(eqcol <= mid, 1.0, 0.0),
                              axis=1, keepdims=True)
                enough = cnt >= rem
                hi2 = jnp.where(enough, mid, hi2)
                lo2 = jnp.where(enough, lo2, mid)

            selected = gt | ((eqcol <= hi2) & (rem > 0))
            bias = jnp.where(selected, 0.0, NEG)

        q = q_ref[...]
        ku = kup_ref[...]
        vu = vup_ref[...]
        ctx = []
        for h in range(NUM_HEADS):
            sl = slice(h * D_HEAD, (h + 1) * D_HEAD)
            s = _dot(q[:, sl], ku[:, sl], trans_b=True) + bias
            e = jnp.exp(s)
            denom = jnp.sum(e, axis=1, keepdims=True)
            cu = _dot(e.astype(jnp.bfloat16), vu[:, sl])
            ctx.append(cu * (1.0 / denom))
        ctx = jnp.concatenate(ctx, axis=1)
        out_ref[...] = (_dot(ctx.astype(jnp.bfloat16), wout_ref[...])
                        + bout_ref[...])

    return attn_kernel


def kernel(x, Wq, bq, Wkv_down, bkv_down, Wk_up, bk_up, Wv_up, bv_up,
           Wq_idx, bq_idx, Wout, bout):
    b, s, dm = x.shape
    x2 = x.reshape(s, dm)
    Wq = Wq.astype(jnp.bfloat16)
    Wk_up = Wk_up.astype(jnp.bfloat16)
    Wv_up = Wv_up.astype(jnp.bfloat16)
    Wout = Wout.astype(jnp.bfloat16)
    row_blk = lambda i: (i, 0)
    whole = lambda i: (0, 0)

    def full_spec(arr):
        return pl.BlockSpec(arr.shape, whole)

    b2 = lambda v: v.reshape(1, -1)

    q, qidx, kdown, kup, vup = pl.pallas_call(
        _proj_kernel,
        grid=(s // BM,),
        in_specs=[
            pl.BlockSpec((BM, dm), row_blk),
            full_spec(Wq), pl.BlockSpec((1, dm), whole),
            full_spec(Wkv_down), pl.BlockSpec((1, 2 * D_LATENT), whole),
            full_spec(Wk_up), pl.BlockSpec((1, dm), whole),
            full_spec(Wv_up), pl.BlockSpec((1, dm), whole),
            full_spec(Wq_idx), pl.BlockSpec((1, D_LATENT), whole),
        ],
        out_specs=[
            pl.BlockSpec((BM, dm), row_blk),
            pl.BlockSpec((BM, D_LATENT), row_blk),
            pl.BlockSpec((BM, D_LATENT), row_blk),
            pl.BlockSpec((BM, dm), row_blk),
            pl.BlockSpec((BM, dm), row_blk),
        ],
        out_shape=[
            jax.ShapeDtypeStruct((s, dm), jnp.bfloat16),
            jax.ShapeDtypeStruct((s, D_LATENT), jnp.float32),
            jax.ShapeDtypeStruct((s, D_LATENT), jnp.float32),
            jax.ShapeDtypeStruct((s, dm), jnp.bfloat16),
            jax.ShapeDtypeStruct((s, dm), jnp.bfloat16),
        ],
    )(x2, Wq, b2(bq), Wkv_down, b2(bkv_down), Wk_up, b2(bk_up),
      Wv_up, b2(bv_up), Wq_idx, b2(bq_idx))

    outs = []
    bm2 = 512
    for g in range(4):
        row_base = g * bm2
        s_kv = (g + 1) * bm2
        grp_blk = lambda i, g=g: (i + g, 0)
        out_g = pl.pallas_call(
            _make_attn_kernel(s_kv, row_base, bm2),
            grid=(1,),
            in_specs=[
                pl.BlockSpec((bm2, D_LATENT), grp_blk),
                pl.BlockSpec((s_kv, D_LATENT), whole),
                pl.BlockSpec((bm2, dm), grp_blk),
                pl.BlockSpec((s_kv, dm), whole),
                pl.BlockSpec((s_kv, dm), whole),
                full_spec(Wout), pl.BlockSpec((1, dm), whole),
            ],
            out_specs=pl.BlockSpec((bm2, dm), row_blk),
            out_shape=jax.ShapeDtypeStruct((bm2, dm), jnp.float32),
        )(qidx, kdown, q, kup, vup, Wout, b2(bout))
        outs.append(out_g)

    return jnp.concatenate(outs, axis=0).reshape(b, s, dm)
